# baseline (speedup 1.0000x reference)
import functools

import jax
import jax.numpy as jnp
from jax import lax
from jax.experimental import pallas as pl
from jax.experimental.pallas import tpu as pltpu
from jax.experimental.pallas import tpu_sc as plsc

N_NODES = 10000
N_EDGES = 320000
IN_DIM = 128
HID_DIM = 32

NC = 2
NS = 16
NW = NC * NS
CHUNK = 128
NPAD = 10240
ROWS_PER_TILE = NPAD // NS
TCH = 79
NCH = NW * TCH
EPAD = NCH * CHUNK
TCH0 = 63
TCH1 = 95
TCHMAX = max(TCH0, TCH1)
CW = 8
NBUF = 4

_f32 = jnp.float32


def _sc_count_body(dst_hbm, ones_hbm, zeros_hbm, cnt_hbm,
                   idx_v, ones_v, rb_v, acc_sh):
    c = lax.axis_index("c")
    s = lax.axis_index("s")
    wid = c * NS + s
    base_row = s * ROWS_PER_TILE

    pltpu.sync_copy(zeros_hbm, rb_v.at[pl.ds(0, CHUNK)])
    for r in range(ROWS_PER_TILE // CHUNK):
        pltpu.sync_copy(rb_v.at[pl.ds(0, CHUNK)],
                        acc_sh.at[pl.ds(base_row + r * CHUNK, CHUNK)])
    pltpu.sync_copy(ones_hbm, ones_v)
    pltpu.sync_copy(dst_hbm.at[pl.ds(wid * TCH, TCH)], idx_v)
    plsc.subcore_barrier()

    def body(j, carry):
        pltpu.sync_copy(ones_v, acc_sh.at[idx_v.at[j]], add=True)
        return carry

    lax.fori_loop(0, TCH, body, 0)
    plsc.subcore_barrier()

    pltpu.sync_copy(acc_sh.at[pl.ds(base_row, ROWS_PER_TILE)], rb_v)
    pltpu.sync_copy(rb_v, cnt_hbm.at[c, pl.ds(base_row, ROWS_PER_TILE)])


@functools.lru_cache(maxsize=None)
def _sc_count():
    return pl.kernel(
        _sc_count_body,
        out_type=jax.ShapeDtypeStruct((NC, NPAD, CW), _f32),
        mesh=plsc.VectorSubcoreMesh(core_axis_name="c",
                                    subcore_axis_name="s"),
        scratch_types=[
            pltpu.VMEM((TCH, CHUNK), jnp.int32),
            pltpu.VMEM((CHUNK, CW), _f32),
            pltpu.VMEM((ROWS_PER_TILE, CW), _f32),
            pltpu.VMEM_SHARED((NPAD, CW), _f32),
        ],
        compiler_params=pltpu.CompilerParams(use_tc_tiling_on_sc=False),
    )


def _sc_agg_body(table_hbm, src_hbm, dst_hbm, zeros_hbm, acc_hbm,
                 sidx_v, didx_v, r0, r1, r2, r3, rb_v,
                 sg0, sg1, sg2, sg3, ss0, ss1, ss2, ss3, acc_sh):
    rows = (r0, r1, r2, r3)
    semg = (sg0, sg1, sg2, sg3)
    sems = (ss0, ss1, ss2, ss3)
    c = lax.axis_index("c")
    s = lax.axis_index("s")
    wid = c * NS + s
    base_row = s * ROWS_PER_TILE

    pltpu.sync_copy(zeros_hbm, r0)
    for r in range(ROWS_PER_TILE // CHUNK):
        pltpu.sync_copy(r0, acc_sh.at[pl.ds(base_row + r * CHUNK, CHUNK)])
    plsc.subcore_barrier()

    def gath(cc, b):
        pltpu.async_copy(table_hbm.at[sidx_v.at[cc]], rows[b], semg[b])

    def wait_g(cc, b):
        pltpu.make_async_copy(table_hbm.at[sidx_v.at[cc]], rows[b],
                              semg[b]).wait()

    def scat(cc, b):
        pltpu.async_copy(rows[b], acc_sh.at[didx_v.at[cc]], sems[b],
                         add=True)

    def wait_s(cc, b):
        pltpu.make_async_copy(rows[b], acc_sh.at[didx_v.at[cc]],
                              sems[b]).wait()

    def pipeline(tch, chunk_base):
        pltpu.sync_copy(src_hbm.at[pl.ds(chunk_base, tch)],
                        sidx_v.at[pl.ds(0, tch)])
        pltpu.sync_copy(dst_hbm.at[pl.ds(chunk_base, tch)],
                        didx_v.at[pl.ds(0, tch)])
        gath(0, 0)
        gath(1, 1)
        gath(2, 2)
        wait_g(0, 0)
        scat(0, 0)
        gath(3, 3)
        wait_g(1, 1)
        scat(1, 1)

        def body(j, carry):
            for b in range(NBUF):
                cc = j * NBUF + b
                wait_s(cc - 4, b)
                gath(cc, b)
                bp = (b + 2) % NBUF
                wait_g(cc - 2, bp)
                scat(cc - 2, bp)
            return carry

        lax.fori_loop(1, (tch - 4) // NBUF + 1, body, 0)

        for cc in range(NBUF * ((tch - 4) // NBUF + 1), tch):
            b = cc % NBUF
            wait_s(cc - 4, b)
            gath(cc, b)
            bp = (cc - 2) % NBUF
            wait_g(cc - 2, bp)
            scat(cc - 2, bp)
        for cc in range(tch - 2, tch):
            wait_g(cc, cc % NBUF)
            scat(cc, cc % NBUF)
        for cc in range(tch - 4, tch):
            wait_s(cc, cc % NBUF)

    @pl.when(c == 0)
    def _():
        pipeline(TCH0, s * TCH0)

    @pl.when(c == 1)
    def _():
        pipeline(TCH1, NS * TCH0 + s * TCH1)

    plsc.subcore_barrier()

    pltpu.sync_copy(acc_sh.at[pl.ds(base_row, ROWS_PER_TILE)], rb_v)
    pltpu.sync_copy(rb_v, acc_hbm.at[c, pl.ds(base_row, ROWS_PER_TILE)])


@functools.lru_cache(maxsize=None)
def _sc_agg():
    return pl.kernel(
        _sc_agg_body,
        out_type=jax.ShapeDtypeStruct((NC, NPAD, HID_DIM), _f32),
        mesh=plsc.VectorSubcoreMesh(core_axis_name="c",
                                    subcore_axis_name="s"),
        scratch_types=[
            pltpu.VMEM((TCHMAX, CHUNK), jnp.int32),
            pltpu.VMEM((TCHMAX, CHUNK), jnp.int32),
            pltpu.VMEM((CHUNK, HID_DIM), _f32),
            pltpu.VMEM((CHUNK, HID_DIM), _f32),
            pltpu.VMEM((CHUNK, HID_DIM), _f32),
            pltpu.VMEM((CHUNK, HID_DIM), _f32),
            pltpu.VMEM((ROWS_PER_TILE, HID_DIM), _f32),
            pltpu.SemaphoreType.DMA,
            pltpu.SemaphoreType.DMA,
            pltpu.SemaphoreType.DMA,
            pltpu.SemaphoreType.DMA,
            pltpu.SemaphoreType.DMA,
            pltpu.SemaphoreType.DMA,
            pltpu.SemaphoreType.DMA,
            pltpu.SemaphoreType.DMA,
            pltpu.VMEM_SHARED((NPAD, HID_DIM), _f32),
        ],
        compiler_params=pltpu.CompilerParams(use_tc_tiling_on_sc=False),
    )


_BLK = 512
_GRID = NPAD // _BLK


def _tc_mm_body(x_ref, w_ref, h_ref):
    h_ref[...] = jnp.dot(x_ref[...], w_ref[...],
                         preferred_element_type=_f32,
                         precision=lax.Precision.HIGHEST)


def _tc_mm(x, w1):
    return pl.pallas_call(
        _tc_mm_body,
        grid=(_GRID,),
        in_specs=[
            pl.BlockSpec((_BLK, IN_DIM), lambda i: (i, 0)),
            pl.BlockSpec((IN_DIM, HID_DIM), lambda i: (0, 0)),
        ],
        out_specs=pl.BlockSpec((_BLK, HID_DIM), lambda i: (i, 0)),
        out_shape=jax.ShapeDtypeStruct((NPAD, HID_DIM), _f32),
    )(x, w1)


def _tc_scale_body(h_ref, cnt_ref, xs_ref, dis_ref):
    deg = cnt_ref[0, :, 0:1] + cnt_ref[1, :, 0:1] + 1.0
    dis = lax.rsqrt(deg)
    xs_ref[...] = h_ref[...] * dis
    dis_ref[...] = dis


def _tc_scale(h, cnt):
    return pl.pallas_call(
        _tc_scale_body,
        grid=(_GRID,),
        in_specs=[
            pl.BlockSpec((_BLK, HID_DIM), lambda i: (i, 0)),
            pl.BlockSpec((NC, _BLK, CW), lambda i: (0, i, 0)),
        ],
        out_specs=[
            pl.BlockSpec((_BLK, HID_DIM), lambda i: (i, 0)),
            pl.BlockSpec((_BLK, 1), lambda i: (i, 0)),
        ],
        out_shape=[
            jax.ShapeDtypeStruct((NPAD, HID_DIM), _f32),
            jax.ShapeDtypeStruct((NPAD, 1), _f32),
        ],
    )(h, cnt)


def _tc_glue1_body(acc_ref, xs_ref, dis_ref, b1_ref, ys_ref):
    dis = dis_ref[...]
    agg = dis * (acc_ref[0] + acc_ref[1] + xs_ref[...])
    h1 = jnp.maximum(agg + b1_ref[...], 0.0)
    ys_ref[...] = dis * h1


def _tc_glue1(acc, xs, dis, b1row):
    return pl.pallas_call(
        _tc_glue1_body,
        grid=(_GRID,),
        in_specs=[
            pl.BlockSpec((NC, _BLK, HID_DIM), lambda i: (0, i, 0)),
            pl.BlockSpec((_BLK, HID_DIM), lambda i: (i, 0)),
            pl.BlockSpec((_BLK, 1), lambda i: (i, 0)),
            pl.BlockSpec((1, HID_DIM), lambda i: (0, 0)),
        ],
        out_specs=pl.BlockSpec((_BLK, HID_DIM), lambda i: (i, 0)),
        out_shape=jax.ShapeDtypeStruct((NPAD, HID_DIM), _f32),
    )(acc, xs, dis, b1row)


def _tc_glue2_body(acc_ref, ys_ref, dis_ref, w2_ref, b2_ref, out_ref):
    agg = dis_ref[...] * (acc_ref[0] + acc_ref[1] + ys_ref[...])
    out_ref[...] = (jnp.sum(agg * w2_ref[...], axis=1, keepdims=True)
                    + b2_ref[...])


def _tc_glue2(acc, ys, dis, w2row, b2sq):
    return pl.pallas_call(
        _tc_glue2_body,
        grid=(_GRID,),
        in_specs=[
            pl.BlockSpec((NC, _BLK, HID_DIM), lambda i: (0, i, 0)),
            pl.BlockSpec((_BLK, HID_DIM), lambda i: (i, 0)),
            pl.BlockSpec((_BLK, 1), lambda i: (i, 0)),
            pl.BlockSpec((1, HID_DIM), lambda i: (0, 0)),
            pl.BlockSpec((1, 1), lambda i: (0, 0)),
        ],
        out_specs=pl.BlockSpec((_BLK, 1), lambda i: (i, 0)),
        out_shape=jax.ShapeDtypeStruct((NPAD, 1), _f32),
    )(acc, ys, dis, w2row, b2sq)


def kernel(x, edge_index, W1, b1, W2, b2):
    ei = edge_index.astype(jnp.int32)
    npad_e = EPAD - N_EDGES
    pad_src = jnp.zeros((npad_e,), jnp.int32)
    pad_dst = N_NODES + (jnp.arange(npad_e, dtype=jnp.int32)
                         % (NPAD - N_NODES))
    src2d = jnp.concatenate([ei[0], pad_src]).reshape(EPAD // CHUNK, CHUNK)
    dst2d = jnp.concatenate([ei[1], pad_dst]).reshape(EPAD // CHUNK, CHUNK)

    ones8 = jnp.ones((CHUNK, CW), _f32)
    zeros8 = jnp.zeros((CHUNK, CW), _f32)
    zeros32 = jnp.zeros((CHUNK, HID_DIM), _f32)

    cnt = _sc_count()(dst2d, ones8, zeros8)
    h = _tc_mm(x, W1)
    xs, dis = _tc_scale(h, cnt)
    acc1 = _sc_agg()(xs, src2d, dst2d, zeros32)
    ys = _tc_glue1(acc1, xs, dis, b1.reshape(1, HID_DIM))
    acc2 = _sc_agg()(ys, src2d, dst2d, zeros32)
    out = _tc_glue2(acc2, ys, dis, W2.reshape(1, HID_DIM),
                    b2.reshape(1, 1))
    return out[:N_NODES, 0]


# speedup vs baseline: 1.1532x; 1.1532x over previous
import functools

import jax
import jax.numpy as jnp
from jax import lax
from jax.experimental import pallas as pl
from jax.experimental.pallas import tpu as pltpu
from jax.experimental.pallas import tpu_sc as plsc

N_NODES = 10000
N_EDGES = 320000
IN_DIM = 128
HID_DIM = 32

NC = 2
NS = 16
NW = NC * NS
CHUNK = 128
NPAD = 10240
ROWS_PER_TILE = NPAD // NS
TCH = 79
NCH = NW * TCH
EPAD = NCH * CHUNK
TCH0 = 95
TCH1 = 63
TCHMAX = max(TCH0, TCH1)
CW = 8
NBUF = 4

_f32 = jnp.float32


def _sc_count_body(dst_hbm, ones_hbm, zeros_hbm, cnt_hbm,
                   idx_v, ones_v, rb_v, acc_sh):
    c = lax.axis_index("c")
    s = lax.axis_index("s")
    wid = c * NS + s
    base_row = s * ROWS_PER_TILE

    pltpu.sync_copy(zeros_hbm, rb_v.at[pl.ds(0, CHUNK)])
    for r in range(ROWS_PER_TILE // CHUNK):
        pltpu.sync_copy(rb_v.at[pl.ds(0, CHUNK)],
                        acc_sh.at[pl.ds(base_row + r * CHUNK, CHUNK)])
    pltpu.sync_copy(ones_hbm, ones_v)
    pltpu.sync_copy(dst_hbm.at[pl.ds(wid * TCH, TCH)], idx_v)
    plsc.subcore_barrier()

    def body(j, carry):
        pltpu.sync_copy(ones_v, acc_sh.at[idx_v.at[j]], add=True)
        return carry

    lax.fori_loop(0, TCH, body, 0)
    plsc.subcore_barrier()

    pltpu.sync_copy(acc_sh.at[pl.ds(base_row, ROWS_PER_TILE)], rb_v)
    pltpu.sync_copy(rb_v, cnt_hbm.at[c, pl.ds(base_row, ROWS_PER_TILE)])


@functools.lru_cache(maxsize=None)
def _sc_count():
    return pl.kernel(
        _sc_count_body,
        out_type=jax.ShapeDtypeStruct((NC, NPAD, CW), _f32),
        mesh=plsc.VectorSubcoreMesh(core_axis_name="c",
                                    subcore_axis_name="s"),
        scratch_types=[
            pltpu.VMEM((TCH, CHUNK), jnp.int32),
            pltpu.VMEM((CHUNK, CW), _f32),
            pltpu.VMEM((ROWS_PER_TILE, CW), _f32),
            pltpu.VMEM_SHARED((NPAD, CW), _f32),
        ],
        compiler_params=pltpu.CompilerParams(use_tc_tiling_on_sc=False),
    )


def _sc_agg_body(table_hbm, src_hbm, dst_hbm, zeros_hbm, acc_hbm,
                 sidx_v, didx_v, r0, r1, r2, r3, rb_v,
                 sg0, sg1, sg2, sg3, ss0, ss1, ss2, ss3, acc_sh):
    rows = (r0, r1, r2, r3)
    semg = (sg0, sg1, sg2, sg3)
    sems = (ss0, ss1, ss2, ss3)
    c = lax.axis_index("c")
    s = lax.axis_index("s")
    wid = c * NS + s
    base_row = s * ROWS_PER_TILE

    pltpu.sync_copy(zeros_hbm, r0)
    for r in range(ROWS_PER_TILE // CHUNK):
        pltpu.sync_copy(r0, acc_sh.at[pl.ds(base_row + r * CHUNK, CHUNK)])
    plsc.subcore_barrier()

    def gath(cc, b):
        pltpu.async_copy(table_hbm.at[sidx_v.at[cc]], rows[b], semg[b])

    def wait_g(cc, b):
        pltpu.make_async_copy(table_hbm.at[sidx_v.at[cc]], rows[b],
                              semg[b]).wait()

    def scat(cc, b):
        pltpu.async_copy(rows[b], acc_sh.at[didx_v.at[cc]], sems[b],
                         add=True)

    def wait_s(cc, b):
        pltpu.make_async_copy(rows[b], acc_sh.at[didx_v.at[cc]],
                              sems[b]).wait()

    def pipeline(tch, chunk_base):
        pltpu.sync_copy(src_hbm.at[pl.ds(chunk_base, tch)],
                        sidx_v.at[pl.ds(0, tch)])
        pltpu.sync_copy(dst_hbm.at[pl.ds(chunk_base, tch)],
                        didx_v.at[pl.ds(0, tch)])
        gath(0, 0)
        gath(1, 1)
        gath(2, 2)
        wait_g(0, 0)
        scat(0, 0)
        gath(3, 3)
        wait_g(1, 1)
        scat(1, 1)

        def body(j, carry):
            for b in range(NBUF):
                cc = j * NBUF + b
                wait_s(cc - 4, b)
                gath(cc, b)
                bp = (b + 2) % NBUF
                wait_g(cc - 2, bp)
                scat(cc - 2, bp)
            return carry

        lax.fori_loop(1, (tch - 4) // NBUF + 1, body, 0)

        for cc in range(NBUF * ((tch - 4) // NBUF + 1), tch):
            b = cc % NBUF
            wait_s(cc - 4, b)
            gath(cc, b)
            bp = (cc - 2) % NBUF
            wait_g(cc - 2, bp)
            scat(cc - 2, bp)
        for cc in range(tch - 2, tch):
            wait_g(cc, cc % NBUF)
            scat(cc, cc % NBUF)
        for cc in range(tch - 4, tch):
            wait_s(cc, cc % NBUF)

    @pl.when(c == 0)
    def _():
        pipeline(TCH0, s * TCH0)

    @pl.when(c == 1)
    def _():
        pipeline(TCH1, NS * TCH0 + s * TCH1)

    plsc.subcore_barrier()

    pltpu.sync_copy(acc_sh.at[pl.ds(base_row, ROWS_PER_TILE)], rb_v)
    pltpu.sync_copy(rb_v, acc_hbm.at[c, pl.ds(base_row, ROWS_PER_TILE)])


@functools.lru_cache(maxsize=None)
def _sc_agg(w):
    return pl.kernel(
        _sc_agg_body,
        out_type=jax.ShapeDtypeStruct((NC, NPAD, w), _f32),
        mesh=plsc.VectorSubcoreMesh(core_axis_name="c",
                                    subcore_axis_name="s"),
        scratch_types=[
            pltpu.VMEM((TCHMAX, CHUNK), jnp.int32),
            pltpu.VMEM((TCHMAX, CHUNK), jnp.int32),
            pltpu.VMEM((CHUNK, w), _f32),
            pltpu.VMEM((CHUNK, w), _f32),
            pltpu.VMEM((CHUNK, w), _f32),
            pltpu.VMEM((CHUNK, w), _f32),
            pltpu.VMEM((ROWS_PER_TILE, w), _f32),
            pltpu.SemaphoreType.DMA,
            pltpu.SemaphoreType.DMA,
            pltpu.SemaphoreType.DMA,
            pltpu.SemaphoreType.DMA,
            pltpu.SemaphoreType.DMA,
            pltpu.SemaphoreType.DMA,
            pltpu.SemaphoreType.DMA,
            pltpu.SemaphoreType.DMA,
            pltpu.VMEM_SHARED((NPAD, w), _f32),
        ],
        compiler_params=pltpu.CompilerParams(use_tc_tiling_on_sc=False),
    )


_BLK = 512
_GRID = NPAD // _BLK


def _tc_mm_body(x_ref, w_ref, h_ref):
    h_ref[...] = jnp.dot(x_ref[...], w_ref[...],
                         preferred_element_type=_f32,
                         precision=lax.Precision.HIGHEST)


def _tc_mm(x, w1):
    return pl.pallas_call(
        _tc_mm_body,
        grid=(_GRID,),
        in_specs=[
            pl.BlockSpec((_BLK, IN_DIM), lambda i: (i, 0)),
            pl.BlockSpec((IN_DIM, HID_DIM), lambda i: (0, 0)),
        ],
        out_specs=pl.BlockSpec((_BLK, HID_DIM), lambda i: (i, 0)),
        out_shape=jax.ShapeDtypeStruct((NPAD, HID_DIM), _f32),
    )(x, w1)


def _tc_scale_body(h_ref, cnt_ref, xs_ref, dis_ref):
    deg = cnt_ref[0, :, 0:1] + cnt_ref[1, :, 0:1] + 1.0
    dis = lax.rsqrt(deg)
    xs_ref[...] = h_ref[...] * dis
    dis_ref[...] = dis


def _tc_scale(h, cnt):
    return pl.pallas_call(
        _tc_scale_body,
        grid=(_GRID,),
        in_specs=[
            pl.BlockSpec((_BLK, HID_DIM), lambda i: (i, 0)),
            pl.BlockSpec((NC, _BLK, CW), lambda i: (0, i, 0)),
        ],
        out_specs=[
            pl.BlockSpec((_BLK, HID_DIM), lambda i: (i, 0)),
            pl.BlockSpec((_BLK, 1), lambda i: (i, 0)),
        ],
        out_shape=[
            jax.ShapeDtypeStruct((NPAD, HID_DIM), _f32),
            jax.ShapeDtypeStruct((NPAD, 1), _f32),
        ],
    )(h, cnt)


def _tc_glue1_body(acc_ref, xs_ref, dis_ref, b1_ref, w2_ref,
                   g8_ref, gs_ref):
    dis = dis_ref[...]
    agg = dis * (acc_ref[0] + acc_ref[1] + xs_ref[...])
    h1 = jnp.maximum(agg + b1_ref[...], 0.0)
    g = jnp.sum(h1 * w2_ref[...], axis=1, keepdims=True)
    gs = dis * g
    g8_ref[...] = gs * jnp.ones((1, CW), _f32)
    gs_ref[...] = gs


def _tc_glue1(acc, xs, dis, b1row, w2row):
    return pl.pallas_call(
        _tc_glue1_body,
        grid=(_GRID,),
        in_specs=[
            pl.BlockSpec((NC, _BLK, HID_DIM), lambda i: (0, i, 0)),
            pl.BlockSpec((_BLK, HID_DIM), lambda i: (i, 0)),
            pl.BlockSpec((_BLK, 1), lambda i: (i, 0)),
            pl.BlockSpec((1, HID_DIM), lambda i: (0, 0)),
            pl.BlockSpec((1, HID_DIM), lambda i: (0, 0)),
        ],
        out_specs=[
            pl.BlockSpec((_BLK, CW), lambda i: (i, 0)),
            pl.BlockSpec((_BLK, 1), lambda i: (i, 0)),
        ],
        out_shape=[
            jax.ShapeDtypeStruct((NPAD, CW), _f32),
            jax.ShapeDtypeStruct((NPAD, 1), _f32),
        ],
    )(acc, xs, dis, b1row, w2row)


def _tc_glue2_body(acc_ref, gs_ref, dis_ref, b2_ref, out_ref):
    out_ref[...] = (dis_ref[...] * (acc_ref[0, :, 0:1] + acc_ref[1, :, 0:1]
                                    + gs_ref[...]) + b2_ref[...])


def _tc_glue2(acc, gs, dis, b2sq):
    return pl.pallas_call(
        _tc_glue2_body,
        grid=(_GRID,),
        in_specs=[
            pl.BlockSpec((NC, _BLK, CW), lambda i: (0, i, 0)),
            pl.BlockSpec((_BLK, 1), lambda i: (i, 0)),
            pl.BlockSpec((_BLK, 1), lambda i: (i, 0)),
            pl.BlockSpec((1, 1), lambda i: (0, 0)),
        ],
        out_specs=pl.BlockSpec((_BLK, 1), lambda i: (i, 0)),
        out_shape=jax.ShapeDtypeStruct((NPAD, 1), _f32),
    )(acc, gs, dis, b2sq)


def kernel(x, edge_index, W1, b1, W2, b2):
    ei = edge_index.astype(jnp.int32)
    npad_e = EPAD - N_EDGES
    pad_src = jnp.zeros((npad_e,), jnp.int32)
    pad_dst = N_NODES + (jnp.arange(npad_e, dtype=jnp.int32)
                         % (NPAD - N_NODES))
    src2d = jnp.concatenate([ei[0], pad_src]).reshape(EPAD // CHUNK, CHUNK)
    dst2d = jnp.concatenate([ei[1], pad_dst]).reshape(EPAD // CHUNK, CHUNK)

    ones8 = jnp.ones((CHUNK, CW), _f32)
    zeros8 = jnp.zeros((CHUNK, CW), _f32)
    zeros32 = jnp.zeros((CHUNK, HID_DIM), _f32)

    cnt = _sc_count()(dst2d, ones8, zeros8)
    h = _tc_mm(x, W1)
    xs, dis = _tc_scale(h, cnt)
    acc1 = _sc_agg(HID_DIM)(xs, src2d, dst2d, zeros32)
    g8, gs = _tc_glue1(acc1, xs, dis, b1.reshape(1, HID_DIM),
                       W2.reshape(1, HID_DIM))
    acc2 = _sc_agg(CW)(g8, src2d, dst2d, zeros8)
    out = _tc_glue2(acc2, gs, dis, b2.reshape(1, 1))
    return out[:N_NODES, 0]
